# baseline (speedup 1.0000x reference)
import functools

import jax
import jax.numpy as jnp
from jax import lax
from jax.experimental import pallas as pl
from jax.experimental.pallas import tpu as pltpu
from jax.experimental.pallas import tpu_sc as plsc

SEQ = 200
BATCH = 4096
EMB = 64
LANES = 16
NCORES = 2
NWORKERS = NCORES * 16
B_PER_W = BATCH // NWORKERS
DCHUNKS = EMB // LANES
INV_SEQ = 1.0 / SEQ


def _pool_body(
    review_hbm, table_hbm, out_hbm, idx_v, idxp, offp, buf0, buf1, acc_t, sem0, sem1
):
    wid = lax.axis_index("s") * NCORES + lax.axis_index("c")
    base = wid * B_PER_W

    pltpu.sync_copy(review_hbm.at[:, pl.ds(base, B_PER_W)], idx_v)

    zeros = jnp.zeros((LANES,), jnp.float32)

    def pbody(s, _):
        for k in range(B_PER_W // LANES):
            sl = pl.ds(k * LANES, LANES)
            i = idx_v[s, sl]
            g = lax.shift_right_logical(
                lax.shift_right_logical(i, 6) * 33555, 22
            )
            t = i - g * _TXB
            h = 1 + lax.shift_right_arithmetic(t - _TXB // 2, 31)
            idxp[s, sl] = g * (_TXB // 2) + t - h * (_TXB // 2)
            offp[s, sl] = h * EMB
        return 0

    lax.fori_loop(0, SEQ, pbody, 0)

    def zbody(e, _):
        for k in range(B_PER_W // LANES):
            acc_t[e, pl.ds(k * LANES, LANES)] = zeros
        return 0

    lax.fori_loop(0, EMB, zbody, 0)

    bufs = (buf0, buf1)
    sems = (sem0, sem1)

    def start(s, par):
        pltpu.async_copy(table_hbm.at[idxp.at[s]], bufs[par], sems[par])

    def wait(par):
        pltpu.make_async_copy(
            table_hbm.at[pl.ds(0, B_PER_W)], bufs[par], sems[par]
        ).wait()

    iota16 = lax.iota(jnp.int32, LANES)

    def accumulate(s, par):
        buf = bufs[par]

        def gbody(k, _):
            rows = iota16 + k * LANES
            offs = offp[s, pl.ds(k * LANES, LANES)]
            for e in range(EMB):
                v = plsc.load_gather(buf, [rows, offs + e])
                plsc.addupdate(acc_t.at[e, pl.ds(k * LANES, LANES)], v)
            return 0

        lax.fori_loop(0, B_PER_W // LANES, gbody, 0)

    start(0, 0)

    def outer(t, _):
        start(2 * t + 1, 1)
        wait(0)
        accumulate(2 * t, 0)
        @pl.when(t < SEQ // 2 - 1)
        def _():
            start(2 * t + 2, 0)

        wait(1)
        accumulate(2 * t + 1, 1)
        return 0

    lax.fori_loop(0, SEQ // 2, outer, 0)

    def sbody(e, _):
        for k in range(B_PER_W // LANES):
            sl = pl.ds(k * LANES, LANES)
            acc_t[e, sl] = acc_t[e, sl] * INV_SEQ
        return 0

    lax.fori_loop(0, EMB, sbody, 0)
    pltpu.sync_copy(acc_t, out_hbm.at[:, pl.ds(base, B_PER_W)])


def _pooled_embedding(review, table):
    mesh = plsc.VectorSubcoreMesh(
        core_axis_name="c", subcore_axis_name="s", num_cores=NCORES
    )
    k = functools.partial(
        pl.kernel,
        mesh=mesh,
        out_type=jax.ShapeDtypeStruct((EMB, BATCH), jnp.float32),
        scratch_types=[
            pltpu.VMEM((SEQ, B_PER_W), jnp.int32),
            pltpu.VMEM((SEQ, B_PER_W), jnp.int32),
            pltpu.VMEM((SEQ, B_PER_W), jnp.int32),
            pltpu.VMEM((B_PER_W, 2 * EMB), jnp.float32),
            pltpu.VMEM((B_PER_W, 2 * EMB), jnp.float32),
            pltpu.VMEM((EMB, B_PER_W), jnp.float32),
            pltpu.SemaphoreType.DMA,
            pltpu.SemaphoreType.DMA,
        ],
        compiler_params=pltpu.CompilerParams(needs_layout_passes=False),
    )(_pool_body)
    return k(review, table)


_TXB = 8000


_NTX = 125


def _tx_body(t_hbm, o_ref, vb0, vb1, sem0, sem1):
    g = pl.program_id(0)
    vocab = t_hbm.shape[1]
    win = _TXB + 192
    vbs = (vb0, vb1)
    sems = (sem0, sem1)

    def win_start(k):
        b = k * _TXB
        s = jnp.minimum(b - b % 128, vocab - win)
        return pl.multiple_of(s, 128)

    def fetch(k, par):
        pltpu.async_copy(t_hbm.at[:, pl.ds(win_start(k), win)], vbs[par], sems[par])

    def process(par):
        pltpu.make_async_copy(
            t_hbm.at[:, pl.ds(0, win)], vbs[par], sems[par]
        ).wait()
        x = vbs[par][...]
        eye = jnp.eye(EMB, dtype=jnp.float32)
        y = jax.lax.dot_general(
            x, eye, (((0,), (0,)), ((), ())),
            preferred_element_type=jnp.float32,
        )
        off = g * _TXB - win_start(g)
        ys = jnp.where(
            off == 0,
            y[:_TXB],
            jnp.where(off == 64, y[64 : _TXB + 64], y[192 : _TXB + 192]),
        )
        o_ref[...] = jnp.concatenate(
            [ys[: _TXB // 2], ys[_TXB // 2 :]], axis=1
        )

    @pl.when(g == 0)
    def _():
        fetch(0, 0)

    even = (g % 2) == 0

    @pl.when(even)
    def _():
        @pl.when(g + 1 < _NTX)
        def _():
            fetch(g + 1, 1)

        process(0)

    @pl.when(jnp.logical_not(even))
    def _():
        @pl.when(g + 1 < _NTX)
        def _():
            fetch(g + 1, 0)

        process(1)


def _pack_table(table):
    t_t = jnp.transpose(table)
    vocab = table.shape[0]
    return pl.pallas_call(
        _tx_body,
        grid=(vocab // _TXB,),
        in_specs=[pl.BlockSpec(memory_space=pl.ANY)],
        out_specs=pl.BlockSpec((_TXB // 2, 2 * EMB), lambda i: (i, 0)),
        out_shape=jax.ShapeDtypeStruct((vocab // 2, 2 * EMB), jnp.float32),
        scratch_shapes=[
            pltpu.VMEM((EMB, _TXB + 192), jnp.float32),
            pltpu.VMEM((EMB, _TXB + 192), jnp.float32),
            pltpu.SemaphoreType.DMA,
            pltpu.SemaphoreType.DMA,
        ],
    )(t_t)


def _mlp_body(x_ref, w1_ref, b1_ref, w2_ref, b2_ref, w3_ref, b3_ref, o_ref):
    x = x_ref[...]
    h = jax.lax.dot_general(
        x, w1_ref[...], (((0,), (0,)), ((), ())),
        preferred_element_type=jnp.float32,
    )
    h = jnp.maximum(h + b1_ref[...][None, :], 0.0)
    h = jnp.dot(h, w2_ref[...], preferred_element_type=jnp.float32)
    h = jnp.maximum(h + b2_ref[...][None, :], 0.0)
    logits = jnp.dot(h, w3_ref[...], preferred_element_type=jnp.float32)
    logits = logits + b3_ref[...][None, :]
    m = jnp.max(logits, axis=-1, keepdims=True)
    shifted = logits - m
    lse = jnp.log(jnp.sum(jnp.exp(shifted), axis=-1, keepdims=True))
    o_ref[...] = shifted - lse


def kernel(review, table, W1, b1, W2, b2, W3, b3):
    review = review.astype(jnp.int32)
    pooled = _pooled_embedding(review, _pack_table(table))
    out = pl.pallas_call(
        _mlp_body,
        out_shape=jax.ShapeDtypeStruct((BATCH, W3.shape[1]), jnp.float32),
    )(pooled, W1, b1, W2, b2, W3, b3)
    return out


# speedup vs baseline: 1.9903x; 1.9903x over previous
import functools

import jax
import jax.numpy as jnp
from jax import lax
from jax.experimental import pallas as pl
from jax.experimental.pallas import tpu as pltpu
from jax.experimental.pallas import tpu_sc as plsc

SEQ = 200
BATCH = 4096
EMB = 64
LANES = 16
NCORES = 2
NWORKERS = NCORES * 16
B_PER_W = BATCH // NWORKERS
DCHUNKS = EMB // LANES
INV_SEQ = 1.0 / SEQ


def _pool_body(review_hbm, table_hbm, out_hbm, idx_v, buf0, buf1, acc, sem0, sem1):
    wid = lax.axis_index("s") * NCORES + lax.axis_index("c")
    base = wid * B_PER_W

    pltpu.sync_copy(review_hbm.at[:, pl.ds(base, B_PER_W)], idx_v)

    zeros = jnp.zeros((LANES,), jnp.float32)

    def zbody(r, _):
        for j in range(DCHUNKS):
            acc[r, pl.ds(j * LANES, LANES)] = zeros
        return 0

    lax.fori_loop(0, B_PER_W, zbody, 0)

    bufs = (buf0, buf1)
    sems = (sem0, sem1)

    def start(s, par):
        pltpu.async_copy(table_hbm.at[idx_v.at[s]], bufs[par], sems[par])

    def wait(par):
        pltpu.make_async_copy(
            table_hbm.at[pl.ds(0, B_PER_W)], bufs[par], sems[par]
        ).wait()

    def accumulate(par):
        buf = bufs[par]

        def abody(r, _):
            for j in range(DCHUNKS):
                v = buf[r, pl.ds(j * LANES, LANES)]
                plsc.addupdate(acc.at[r, pl.ds(j * LANES, LANES)], v)
            return 0

        lax.fori_loop(0, B_PER_W, abody, 0)

    start(0, 0)

    def outer(t, _):
        start(2 * t + 1, 1)
        wait(0)
        accumulate(0)
        @pl.when(t < SEQ // 2 - 1)
        def _():
            start(2 * t + 2, 0)

        wait(1)
        accumulate(1)
        return 0

    lax.fori_loop(0, SEQ // 2, outer, 0)

    def sbody(r, _):
        for j in range(DCHUNKS):
            sl = pl.ds(j * LANES, LANES)
            acc[r, sl] = acc[r, sl] * INV_SEQ
        return 0

    lax.fori_loop(0, B_PER_W, sbody, 0)
    pltpu.sync_copy(acc, out_hbm.at[pl.ds(base, B_PER_W)])


def _pooled_embedding(review, table):
    mesh = plsc.VectorSubcoreMesh(
        core_axis_name="c", subcore_axis_name="s", num_cores=NCORES
    )
    k = functools.partial(
        pl.kernel,
        mesh=mesh,
        out_type=jax.ShapeDtypeStruct((BATCH, EMB), jnp.float32),
        scratch_types=[
            pltpu.VMEM((SEQ, B_PER_W), jnp.int32),
            pltpu.VMEM((B_PER_W, 2 * EMB), jnp.float32),
            pltpu.VMEM((B_PER_W, 2 * EMB), jnp.float32),
            pltpu.VMEM((B_PER_W, EMB), jnp.float32),
            pltpu.SemaphoreType.DMA,
            pltpu.SemaphoreType.DMA,
        ],
    )(_pool_body)
    return k(review, table)


def _mlp_body(x_ref, w1_ref, b1_ref, w2_ref, b2_ref, w3_ref, b3_ref, o_ref):
    x = x_ref[...]
    h = jnp.dot(x, w1_ref[...], preferred_element_type=jnp.float32)
    h = jnp.maximum(h + b1_ref[...][None, :], 0.0)
    h = jnp.dot(h, w2_ref[...], preferred_element_type=jnp.float32)
    h = jnp.maximum(h + b2_ref[...][None, :], 0.0)
    logits = jnp.dot(h, w3_ref[...], preferred_element_type=jnp.float32)
    logits = logits + b3_ref[...][None, :]
    m = jnp.max(logits, axis=-1, keepdims=True)
    shifted = logits - m
    lse = jnp.log(jnp.sum(jnp.exp(shifted), axis=-1, keepdims=True))
    o_ref[...] = shifted - lse


def kernel(review, table, W1, b1, W2, b2, W3, b3):
    review = review.astype(jnp.int32)
    tpad = jnp.pad(table, ((0, 0), (0, EMB)))
    pooled = _pooled_embedding(review, tpad)
    out = pl.pallas_call(
        _mlp_body,
        out_shape=jax.ShapeDtypeStruct((BATCH, W3.shape[1]), jnp.float32),
    )(pooled, W1, b1, W2, b2, W3, b3)
    return out


# speedup vs baseline: 2.1104x; 1.0604x over previous
import functools

import jax
import jax.numpy as jnp
from jax import lax
from jax.experimental import pallas as pl
from jax.experimental.pallas import tpu as pltpu
from jax.experimental.pallas import tpu_sc as plsc

SEQ = 200
BATCH = 4096
EMB = 64
LANES = 16
NCORES = 2
NWORKERS = NCORES * 16
B_PER_W = BATCH // NWORKERS
DCHUNKS = EMB // LANES
INV_SEQ = 1.0 / SEQ


def _pool_body(review_hbm, table_hbm, out_hbm, idx_v, buf0, buf1, acc, sem0, sem1):
    wid = lax.axis_index("s") * NCORES + lax.axis_index("c")
    base = wid * B_PER_W

    pltpu.sync_copy(review_hbm.at[:, pl.ds(base, B_PER_W)], idx_v)

    def dbody(s, _):
        for k in range(B_PER_W // LANES):
            sl = pl.ds(k * LANES, LANES)
            idx_v[s, sl] = lax.shift_left(idx_v[s, sl], 1)
        return 0

    lax.fori_loop(0, SEQ, dbody, 0)

    zeros = jnp.zeros((LANES,), jnp.float32)

    def zbody(r, _):
        for j in range(DCHUNKS):
            acc[r, pl.ds(j * LANES, LANES)] = zeros
        return 0

    lax.fori_loop(0, B_PER_W, zbody, 0)

    bufs = (buf0, buf1)
    sems = (sem0, sem1)

    def start(s, par):
        pltpu.async_copy(table_hbm.at[idx_v.at[s]], bufs[par], sems[par])

    def wait(par):
        pltpu.make_async_copy(
            table_hbm.at[pl.ds(0, B_PER_W)], bufs[par], sems[par]
        ).wait()

    def accumulate(par):
        buf = bufs[par]

        def abody(r, _):
            for j in range(DCHUNKS):
                v = buf[r, pl.ds(j * LANES, LANES)]
                plsc.addupdate(acc.at[r, pl.ds(j * LANES, LANES)], v)
            return 0

        lax.fori_loop(0, B_PER_W, abody, 0)

    start(0, 0)

    def outer(t, _):
        start(2 * t + 1, 1)
        wait(0)
        accumulate(0)
        @pl.when(t < SEQ // 2 - 1)
        def _():
            start(2 * t + 2, 0)

        wait(1)
        accumulate(1)
        return 0

    lax.fori_loop(0, SEQ // 2, outer, 0)

    def sbody(r, _):
        for j in range(DCHUNKS):
            sl = pl.ds(j * LANES, LANES)
            acc[r, sl] = acc[r, sl] * INV_SEQ
        return 0

    lax.fori_loop(0, B_PER_W, sbody, 0)
    pltpu.sync_copy(acc, out_hbm.at[pl.ds(base, B_PER_W)])


def _pooled_embedding(review, table):
    mesh = plsc.VectorSubcoreMesh(
        core_axis_name="c", subcore_axis_name="s", num_cores=NCORES
    )
    k = functools.partial(
        pl.kernel,
        mesh=mesh,
        out_type=jax.ShapeDtypeStruct((BATCH, EMB), jnp.float32),
        scratch_types=[
            pltpu.VMEM((SEQ, B_PER_W), jnp.int32),
            pltpu.VMEM((B_PER_W, EMB), jnp.float32),
            pltpu.VMEM((B_PER_W, EMB), jnp.float32),
            pltpu.VMEM((B_PER_W, EMB), jnp.float32),
            pltpu.SemaphoreType.DMA,
            pltpu.SemaphoreType.DMA,
        ],
        compiler_params=pltpu.CompilerParams(use_tc_tiling_on_sc=False),
    )(_pool_body)
    return k(review, table)


def _mlp_body(x_ref, w1_ref, b1_ref, w2_ref, b2_ref, w3_ref, b3_ref, o_ref):
    x = x_ref[...]
    h = jnp.dot(x, w1_ref[...], preferred_element_type=jnp.float32)
    h = jnp.maximum(h + b1_ref[...][None, :], 0.0)
    h = jnp.dot(h, w2_ref[...], preferred_element_type=jnp.float32)
    h = jnp.maximum(h + b2_ref[...][None, :], 0.0)
    logits = jnp.dot(h, w3_ref[...], preferred_element_type=jnp.float32)
    logits = logits + b3_ref[...][None, :]
    m = jnp.max(logits, axis=-1, keepdims=True)
    shifted = logits - m
    lse = jnp.log(jnp.sum(jnp.exp(shifted), axis=-1, keepdims=True))
    o_ref[...] = shifted - lse


def kernel(review, table, W1, b1, W2, b2, W3, b3):
    review = review.astype(jnp.int32)
    tpad = jnp.pad(table, ((0, 0), (0, EMB)))
    pooled = _pooled_embedding(review, tpad.reshape(2 * table.shape[0], EMB))
    out = pl.pallas_call(
        _mlp_body,
        out_shape=jax.ShapeDtypeStruct((BATCH, W3.shape[1]), jnp.float32),
    )(pooled, W1, b1, W2, b2, W3, b3)
    return out
